# baseline (speedup 1.0000x reference)
import functools

import jax
import jax.numpy as jnp
from jax.experimental import pallas as pl
from jax.experimental.pallas import tpu as pltpu

B, S, H = 1, 2048, 1024
E, K, I = 8, 2, 1024
ALPHA, LIMIT = 1.702, 7.0
T = B * S


def _moe_kernel(x_ref, rw_ref, rb_ref, wg_ref, wu_ref, bg_ref, bu_ref,
                wd_ref, bd_ref, out_ref, scores_ref):
    e = pl.program_id(0)

    @pl.when(e == 0)
    def _router():
        x = x_ref[...]
        logits = jnp.dot(x, rw_ref[...], preferred_element_type=jnp.float32)
        logits = logits + rb_ref[...]
        eids = jax.lax.broadcasted_iota(jnp.int32, (T, E), 1)
        v0 = jnp.max(logits, axis=-1, keepdims=True)
        cand0 = jnp.where(logits == v0, eids, E)
        i0 = jnp.min(cand0, axis=-1, keepdims=True)
        masked = jnp.where(eids == i0, -jnp.inf, logits)
        v1 = jnp.max(masked, axis=-1, keepdims=True)
        cand1 = jnp.where(masked == v1, eids, E)
        i1 = jnp.min(cand1, axis=-1, keepdims=True)
        w0 = 1.0 / (1.0 + jnp.exp(v1 - v0))
        w1 = 1.0 - w0
        scores_ref[...] = w0 * (eids == i0) + w1 * (eids == i1)
        out_ref[...] = jnp.zeros_like(out_ref)

    eids = jax.lax.broadcasted_iota(jnp.int32, (T, E), 1)
    sc = jnp.sum(scores_ref[...] * (eids == e), axis=-1, keepdims=True)

    x = x_ref[...]
    gate = jnp.dot(x, wg_ref[0], preferred_element_type=jnp.float32) + bg_ref[0]
    up = jnp.dot(x, wu_ref[0], preferred_element_type=jnp.float32) + bu_ref[0]
    gate = jnp.minimum(gate, LIMIT)
    up = jnp.clip(up, -LIMIT, LIMIT)
    glu = gate * jax.nn.sigmoid(gate * ALPHA)
    act = (up + 1.0) * glu
    y = jnp.dot(act, wd_ref[0], preferred_element_type=jnp.float32) + bd_ref[0]
    out_ref[...] += sc * y


@jax.jit
def kernel(hidden_states, router_weight, router_bias, gate_up_proj,
           gate_up_bias, down_proj, down_bias):
    b, s, h = hidden_states.shape
    x = hidden_states.reshape(-1, h)

    w_g = gate_up_proj[:, :, 0::2]
    w_u = gate_up_proj[:, :, 1::2]
    b_g = gate_up_bias[:, 0::2].reshape(E, 1, I)
    b_u = gate_up_bias[:, 1::2].reshape(E, 1, I)
    b_d = down_bias.reshape(E, 1, H)

    out = pl.pallas_call(
        _moe_kernel,
        grid=(E,),
        in_specs=[
            pl.BlockSpec((T, H), lambda e: (0, 0)),
            pl.BlockSpec((H, E), lambda e: (0, 0)),
            pl.BlockSpec((E,), lambda e: (0,)),
            pl.BlockSpec((1, H, I), lambda e: (e, 0, 0)),
            pl.BlockSpec((1, H, I), lambda e: (e, 0, 0)),
            pl.BlockSpec((1, 1, I), lambda e: (e, 0, 0)),
            pl.BlockSpec((1, 1, I), lambda e: (e, 0, 0)),
            pl.BlockSpec((1, I, H), lambda e: (e, 0, 0)),
            pl.BlockSpec((1, 1, H), lambda e: (e, 0, 0)),
        ],
        out_specs=pl.BlockSpec((T, H), lambda e: (0, 0)),
        out_shape=jax.ShapeDtypeStruct((T, H), jnp.float32),
        scratch_shapes=[pltpu.VMEM((T, E), jnp.float32)],
        compiler_params=pltpu.CompilerParams(
            dimension_semantics=("arbitrary",),
        ),
    )(x, router_weight, router_bias, w_g, w_u, b_g, b_u, down_proj, b_d)

    return out.reshape(b, s, h)
